# Initial kernel scaffold; baseline (speedup 1.0000x reference)
import functools

import jax
import jax.numpy as jnp
from jax import lax
from jax.experimental import pallas as pl
from jax.experimental.pallas import tpu as pltpu
from jax.experimental.pallas import tpu_sc as plsc

N = 10000
E = 320000
D = 128
NS = 16
CH = 80
ROWS_PER_TILE = N // NS
_f32 = jnp.float32


def _mesh():
    return plsc.VectorSubcoreMesh(core_axis_name="c", subcore_axis_name="s")


_DEG_PER_TILE = (E // 2) // NS
_DEG_NCH = _DEG_PER_TILE // CH


def _deg_body(edge_hbm, deg_hbm, ib0, ib1, ones_v, zbuf, deg_sh, is0, is1):
    c = lax.axis_index("c")
    s = lax.axis_index("s")
    ebase = c * (E // 2) + s * _DEG_PER_TILE
    ibuf = (ib0, ib1)
    isem = (is0, is1)

    for k in range(CH // 16):
        ones_v[pl.ds(k * 16, 16)] = jnp.full((16,), 1.0, _f32)

    def zfill(i, carry):
        zbuf[pl.ds(i * 16, 16)] = jnp.zeros((16,), _f32)
        return carry

    lax.fori_loop(0, 2000 // 16, zfill, 0)

    @pl.when(s < 5)
    def _():
        pltpu.sync_copy(zbuf, deg_sh.at[pl.ds(s * 2000, 2000)])

    plsc.subcore_barrier()

    def start_idx(j, bi):
        pltpu.async_copy(
            edge_hbm.at[1, pl.ds(ebase + j * CH, CH)], ibuf[bi], isem[bi])

    def wait_idx(bi):
        pltpu.make_async_copy(
            edge_hbm.at[1, pl.ds(ebase, CH)], ibuf[bi], isem[bi]).wait()

    def scat(bi):
        pltpu.sync_copy(ones_v, deg_sh.at[ibuf[bi]], add=True)

    start_idx(0, 0)

    def pair(jj, carry):
        j0 = jj * 2
        wait_idx(0)
        start_idx(j0 + 1, 1)
        scat(0)
        wait_idx(1)
        start_idx(j0 + 2, 0)
        scat(1)
        return carry

    lax.fori_loop(0, (_DEG_NCH - 1) // 2, pair, 0)
    wait_idx(0)
    scat(0)

    plsc.subcore_barrier()

    @pl.when(s < 5)
    def _():
        pltpu.sync_copy(deg_sh.at[pl.ds(s * 2000, 2000)],
                        deg_hbm.at[c, pl.ds(s * 2000, 2000)])


_deg_call = functools.partial(
    pl.kernel,
    out_type=jax.ShapeDtypeStruct((2, N), _f32),
    mesh=_mesh(),
    scratch_types=[
        pltpu.VMEM((CH,), jnp.int32),
        pltpu.VMEM((CH,), jnp.int32),
        pltpu.VMEM((CH,), _f32),
        pltpu.VMEM((2000,), _f32),
        pltpu.VMEM_SHARED((N,), _f32),
        pltpu.SemaphoreType.DMA,
        pltpu.SemaphoreType.DMA,
    ],
)(_deg_body)


_SC_PER_TILE = E // NS
_SC_NCH = _SC_PER_TILE // CH


def _scat_body(edge_hbm, hs_hbm, acc_hbm,
               sb0, sb1, db0, db1, rb0, rb1, zbuf, acc_sh,
               ss0, ss1, ds0, ds1, gs0, gs1):
    c = lax.axis_index("c")
    s = lax.axis_index("s")
    cN = c * N
    ebase = s * _SC_PER_TILE
    sbuf = (sb0, sb1)
    dbuf = (db0, db1)
    rbuf = (rb0, rb1)
    ssem = (ss0, ss1)
    dsem = (ds0, ds1)
    gsem = (gs0, gs1)

    def zfill(i, carry):
        for j in range(8):
            zbuf[i, pl.ds(j * 16, 16)] = jnp.zeros((16,), _f32)
        return carry

    lax.fori_loop(0, 25, zfill, 0)

    def zcopy(k, carry):
        pltpu.sync_copy(zbuf, acc_sh.at[pl.ds(s * ROWS_PER_TILE + k * 25, 25)])
        return carry

    lax.fori_loop(0, ROWS_PER_TILE // 25, zcopy, 0)
    plsc.subcore_barrier()

    def start_idx(j, bi):
        off = ebase + j * CH
        pltpu.async_copy(edge_hbm.at[0, pl.ds(off, CH)], sbuf[bi], ssem[bi])
        pltpu.async_copy(edge_hbm.at[1, pl.ds(off, CH)], dbuf[bi], dsem[bi])

    def wait_idx(bi):
        pltpu.make_async_copy(
            edge_hbm.at[0, pl.ds(ebase, CH)], sbuf[bi], ssem[bi]).wait()
        pltpu.make_async_copy(
            edge_hbm.at[1, pl.ds(ebase, CH)], dbuf[bi], dsem[bi]).wait()

    def fix_idx(bi):
        for k in range(CH // 16):
            v = sbuf[bi][pl.ds(k * 16, 16)]
            sbuf[bi][pl.ds(k * 16, 16)] = v + cN

    def start_gather(bi):
        pltpu.async_copy(hs_hbm.at[sbuf[bi]], rbuf[bi], gsem[bi])

    def wait_gather(bi):
        pltpu.make_async_copy(hs_hbm.at[sbuf[bi]], rbuf[bi], gsem[bi]).wait()

    def scat(bi):
        pltpu.sync_copy(rbuf[bi], acc_sh.at[dbuf[bi]], add=True)

    def chunk(j, bi, has_next, has_next2):
        wait_gather(bi)
        if has_next:
            wait_idx(1 - bi)
            fix_idx(1 - bi)
            start_gather(1 - bi)
        scat(bi)
        if has_next2:
            start_idx(j + 2, bi)

    start_idx(0, 0)
    start_idx(1, 1)
    wait_idx(0)
    fix_idx(0)
    start_gather(0)

    def pair(jj, carry):
        j0 = jj * 2
        chunk(j0, 0, True, True)
        chunk(j0 + 1, 1, True, True)
        return carry

    lax.fori_loop(0, (_SC_NCH - 2) // 2, pair, 0)
    chunk(_SC_NCH - 2, 0, True, False)
    chunk(_SC_NCH - 1, 1, False, False)

    plsc.subcore_barrier()
    pltpu.sync_copy(acc_sh.at[pl.ds(s * ROWS_PER_TILE, ROWS_PER_TILE)],
                    acc_hbm.at[c, pl.ds(s * ROWS_PER_TILE, ROWS_PER_TILE)])


_scat_call = functools.partial(
    pl.kernel,
    out_type=jax.ShapeDtypeStruct((2, N, D), _f32),
    mesh=_mesh(),
    scratch_types=[
        pltpu.VMEM((CH,), jnp.int32),
        pltpu.VMEM((CH,), jnp.int32),
        pltpu.VMEM((CH,), jnp.int32),
        pltpu.VMEM((CH,), jnp.int32),
        pltpu.VMEM((CH, D), _f32),
        pltpu.VMEM((CH, D), _f32),
        pltpu.VMEM((25, D), _f32),
        pltpu.VMEM_SHARED((N, D), _f32),
        pltpu.SemaphoreType.DMA,
        pltpu.SemaphoreType.DMA,
        pltpu.SemaphoreType.DMA,
        pltpu.SemaphoreType.DMA,
        pltpu.SemaphoreType.DMA,
        pltpu.SemaphoreType.DMA,
    ],
)(_scat_body)


_BLK = 1000
_DN = (((1,), (1,)), ((), ()))


def _prelu(v, a):
    return jnp.where(v >= 0, v, a * v)


def _k1_body(x_ref, lw_ref, lb_ref, a_ref, w1_ref, deg_ref, out_ref):
    a = a_ref[0, 0]
    h0 = lax.dot_general(x_ref[...], lw_ref[...], _DN,
                         preferred_element_type=_f32) + lb_ref[...]
    h0 = _prelu(h0, a)
    h1 = lax.dot_general(h0, w1_ref[...], _DN, preferred_element_type=_f32)
    dinv = lax.rsqrt(deg_ref[0] + deg_ref[1] + 1.0)
    hs = h1 * dinv
    out_ref[0] = hs[:, :D]
    out_ref[1] = hs[:, D:]


def _k3_body(acc_ref, hs_ref, deg_ref, b_ref, a_ref, w_ref, out_ref):
    a = a_ref[0, 0]
    dinv = lax.rsqrt(deg_ref[0] + deg_ref[1] + 1.0)
    m = jnp.concatenate([acc_ref[0] + hs_ref[0], acc_ref[1] + hs_ref[1]],
                        axis=1)
    o = _prelu(m * dinv + b_ref[...], a)
    h = lax.dot_general(o, w_ref[...], _DN, preferred_element_type=_f32)
    hs2 = h * dinv
    out_ref[0] = hs2[:, :D]
    out_ref[1] = hs2[:, D:]


def _k5_body(acc_ref, hs_ref, deg_ref, b_ref, a_ref, out_ref):
    a = a_ref[0, 0]
    dinv = lax.rsqrt(deg_ref[0] + deg_ref[1] + 1.0)
    m = jnp.concatenate([acc_ref[0] + hs_ref[0], acc_ref[1] + hs_ref[1]],
                        axis=1)
    out_ref[...] = _prelu(m * dinv + b_ref[...], a)


def _k1_call(x, lw, lb, a, w1, deg3):
    return pl.pallas_call(
        _k1_body,
        grid=(N // _BLK,),
        in_specs=[
            pl.BlockSpec((_BLK, 128), lambda i: (i, 0)),
            pl.BlockSpec((256, 128), lambda i: (0, 0)),
            pl.BlockSpec((1, 256), lambda i: (0, 0)),
            pl.BlockSpec((1, 1), lambda i: (0, 0)),
            pl.BlockSpec((256, 256), lambda i: (0, 0)),
            pl.BlockSpec((2, _BLK, 1), lambda i: (0, i, 0)),
        ],
        out_specs=pl.BlockSpec((2, _BLK, D), lambda i: (0, i, 0)),
        out_shape=jax.ShapeDtypeStruct((2, N, D), _f32),
    )(x, lw, lb, a, w1, deg3)


def _k3_call(acc, hs, deg3, b, a, w):
    return pl.pallas_call(
        _k3_body,
        grid=(N // _BLK,),
        in_specs=[
            pl.BlockSpec((2, _BLK, D), lambda i: (0, i, 0)),
            pl.BlockSpec((2, _BLK, D), lambda i: (0, i, 0)),
            pl.BlockSpec((2, _BLK, 1), lambda i: (0, i, 0)),
            pl.BlockSpec((1, 256), lambda i: (0, 0)),
            pl.BlockSpec((1, 1), lambda i: (0, 0)),
            pl.BlockSpec((256, 256), lambda i: (0, 0)),
        ],
        out_specs=pl.BlockSpec((2, _BLK, D), lambda i: (0, i, 0)),
        out_shape=jax.ShapeDtypeStruct((2, N, D), _f32),
    )(acc, hs, deg3, b, a, w)


def _k5_call(acc, hs, deg3, b, a):
    return pl.pallas_call(
        _k5_body,
        grid=(N // _BLK,),
        in_specs=[
            pl.BlockSpec((2, _BLK, D), lambda i: (0, i, 0)),
            pl.BlockSpec((2, _BLK, D), lambda i: (0, i, 0)),
            pl.BlockSpec((2, _BLK, 1), lambda i: (0, i, 0)),
            pl.BlockSpec((1, 256), lambda i: (0, 0)),
            pl.BlockSpec((1, 1), lambda i: (0, 0)),
        ],
        out_specs=pl.BlockSpec((_BLK, 256), lambda i: (i, 0)),
        out_shape=jax.ShapeDtypeStruct((N, 256), _f32),
    )(acc, hs, deg3, b, a)


def kernel(x, edge_index, lin_W, lin_b, prelu_a, conv1_W, conv1_b,
           conv2_W, conv2_b):
    edge_index = edge_index.astype(jnp.int32)
    lb = lin_b.reshape(1, 256)
    b1 = conv1_b.reshape(1, 256)
    b2 = conv2_b.reshape(1, 256)
    a = prelu_a.reshape(1, 1)

    deg3 = _deg_call(edge_index)[:, :, None]
    hs1 = _k1_call(x, lin_W, lb, a, conv1_W, deg3)
    acc1 = _scat_call(edge_index, hs1.reshape(2 * N, D))
    hs2 = _k3_call(acc1, hs1, deg3, b1, a, conv2_W)
    acc2 = _scat_call(edge_index, hs2.reshape(2 * N, D))
    return _k5_call(acc2, hs2, deg3, b2, a)


# speedup vs baseline: 17.3578x; 17.3578x over previous
import functools

import jax
import jax.numpy as jnp
from jax import lax
from jax.experimental import pallas as pl
from jax.experimental.pallas import tpu as pltpu
from jax.experimental.pallas import tpu_sc as plsc

N = 10000
NPAD = 10240
E = 320000
D = 128
NS = 16
CH = 80
RPT = NPAD // NS
HOP = 128
_f32 = jnp.float32


def _mesh():
    return plsc.VectorSubcoreMesh(core_axis_name="c", subcore_axis_name="s")


_DEG_PER_TILE = (E // 2) // NS
_DEG_NCH = _DEG_PER_TILE // CH


def _deg_body(dst_hbm, deg_hbm, ib0, ib1, ib2, ib3, ones_v, zbuf, deg_sh,
              is0, is1, is2, is3, as0, as1):
    c = lax.axis_index("c")
    s = lax.axis_index("s")
    ebase = c * (E // 2) + s * _DEG_PER_TILE
    ibuf = (ib0, ib1, ib2, ib3)
    isem = (is0, is1, is2, is3)
    asem = (as0, as1)

    for k in range(CH // 16):
        ones_v[pl.ds(k * 16, 16)] = jnp.full((16,), 1.0, _f32)

    def zfill(i, carry):
        zbuf[pl.ds(i * 16, 16)] = jnp.zeros((16,), _f32)
        return carry

    lax.fori_loop(0, 2000 // 16, zfill, 0)

    def start_idx(j, q4):
        pltpu.async_copy(
            dst_hbm.at[pl.ds(ebase + j * CH, CH)], ibuf[q4], isem[q4])

    def wait_idx(q4):
        pltpu.make_async_copy(
            dst_hbm.at[pl.ds(ebase, CH)], ibuf[q4], isem[q4]).wait()

    def start_scat(q4, q2):
        pltpu.async_copy(ones_v, deg_sh.at[ibuf[q4]], asem[q2], add=True)

    def wait_scat(q4, q2):
        pltpu.make_async_copy(ones_v, deg_sh.at[ibuf[q4]], asem[q2]).wait()

    for j in range(2):
        start_idx(j, j)

    @pl.when(s < 5)
    def _():
        pltpu.sync_copy(zbuf, deg_sh.at[pl.ds(s * 2000, 2000)])

    plsc.subcore_barrier()

    def chunk(j, q4, first2, idx_ahead):
        q2 = q4 % 2
        wait_idx(q4)
        if not first2:
            wait_scat((q4 + 2) % 4, q2)
        if idx_ahead:
            start_idx(j + 2, (q4 + 2) % 4)
        start_scat(q4, q2)

    chunk(0, 0, True, True)
    chunk(1, 1, True, True)

    def group(g, carry):
        j0 = 2 + 4 * g
        for k in range(4):
            chunk(j0 + k, (2 + k) % 4, False, True)
        return carry

    lax.fori_loop(0, 30, group, 0)
    chunk(122, 122 % 4, False, True)
    chunk(123, 123 % 4, False, False)
    chunk(124, 124 % 4, False, False)
    for j in range(123, 125):
        wait_scat(j % 4, j % 2)

    plsc.subcore_barrier()

    @pl.when(s < 5)
    def _():
        pltpu.sync_copy(deg_sh.at[pl.ds(s * 2000, 2000)], zbuf)
        pltpu.sync_copy(zbuf, deg_hbm.at[pl.ds(c * N + s * 2000, 2000)])


@functools.cache
def _deg_kernel():
    return pl.kernel(
        _deg_body,
        out_type=jax.ShapeDtypeStruct((2 * N,), _f32),
        mesh=_mesh(),
        scratch_types=(
            [pltpu.VMEM((CH,), jnp.int32)] * 4
            + [pltpu.VMEM((CH,), _f32)]
            + [pltpu.VMEM((2000,), _f32)]
            + [pltpu.VMEM_SHARED((N,), _f32)]
            + [pltpu.SemaphoreType.DMA] * 6
        ),
    )


def _deg_call(dst):
    return _deg_kernel()(dst)


_SC_PER_TILE = E // NS
_SC_NCH = _SC_PER_TILE // CH


def _scat_body(src_hbm, dst_hbm, hs_hbm, acc_hbm,
               sb0, sb1, sb2, sb3, sb4, sb5, sb6, sb7,
               db0, db1, db2, db3, db4, db5, db6, db7,
               rb0, rb1, rb2, rb3, acc_sh,
               is0, is1, is2, is3, is4, is5, is6, is7,
               gs0, gs1, gs2, gs3, as0, as1, as2, as3,
               ws0, ws1, ws2, ws3):
    c = lax.axis_index("c")
    s = lax.axis_index("s")
    cN = c * N
    ebase = s * _SC_PER_TILE
    sbuf = (sb0, sb1, sb2, sb3, sb4, sb5, sb6, sb7)
    dbuf = (db0, db1, db2, db3, db4, db5, db6, db7)
    rbuf = (rb0, rb1, rb2, rb3)
    isem = (is0, is1, is2, is3, is4, is5, is6, is7)
    gsem = (gs0, gs1, gs2, gs3)
    asem = (as0, as1, as2, as3)
    wsem = (ws0, ws1, ws2, ws3)
    NHOP = RPT // CH

    def start_idx(j, q8):
        off = ebase + j * CH
        pltpu.async_copy(src_hbm.at[pl.ds(off, CH)], sbuf[q8], isem[q8])
        pltpu.async_copy(dst_hbm.at[pl.ds(off, CH)], dbuf[q8], isem[q8])

    def wait_idx(q8):
        pltpu.make_async_copy(
            src_hbm.at[pl.ds(ebase, CH)], sbuf[q8], isem[q8]).wait()
        pltpu.make_async_copy(
            dst_hbm.at[pl.ds(ebase, CH)], dbuf[q8], isem[q8]).wait()

    def fix_idx(q8):
        for k in range(CH // 16):
            v = sbuf[q8][pl.ds(k * 16, 16)]
            sbuf[q8][pl.ds(k * 16, 16)] = v + cN

    def start_gather(q8, q4):
        pltpu.async_copy(hs_hbm.at[sbuf[q8]], rbuf[q4], gsem[q4])

    def wait_gather(q8, q4):
        pltpu.make_async_copy(hs_hbm.at[sbuf[q8]], rbuf[q4], gsem[q4]).wait()

    def start_scat(q8, q4):
        pltpu.async_copy(rbuf[q4], acc_sh.at[dbuf[q8]], asem[q4], add=True)

    def wait_scat(q8, q4):
        pltpu.make_async_copy(
            rbuf[q4], acc_sh.at[dbuf[q8]], asem[q4]).wait()

    for j in range(5):
        start_idx(j, j)

    def zfill(i, carry):
        for j in range(8):
            rb0[i, pl.ds(j * 16, 16)] = jnp.zeros((16,), _f32)
        return carry

    lax.fori_loop(0, CH, zfill, 0)
    for k in range(NHOP):
        pltpu.async_copy(rb0, acc_sh.at[pl.ds(s * RPT + k * CH, CH)],
                         wsem[0])
    for k in range(NHOP):
        pltpu.make_async_copy(
            rb0, acc_sh.at[pl.ds(s * RPT, CH)], wsem[0]).wait()

    wait_idx(0)
    fix_idx(0)
    start_gather(0, 0)
    plsc.subcore_barrier()

    def chunk(j, q8, first3, has_next, idx_ahead):
        q4 = q8 % 4
        wait_gather(q8, q4)
        start_scat(q8, q4)
        if has_next:
            wait_idx((q8 + 1) % 8)
            fix_idx((q8 + 1) % 8)
            if not first3:
                wait_scat((q8 + 5) % 8, (q8 + 1) % 4)
            start_gather((q8 + 1) % 8, (q8 + 1) % 4)
        if idx_ahead:
            start_idx(j + 5, (q8 + 5) % 8)

    chunk(0, 0, True, True, True)
    chunk(1, 1, True, True, True)
    chunk(2, 2, True, True, True)

    def group(g, carry):
        j0 = 3 + 8 * g
        for k in range(8):
            chunk(j0 + k, (3 + k) % 8, False, True, True)
        return carry

    lax.fori_loop(0, 30, group, 0)
    for j in range(243, 250):
        chunk(j, j % 8, False, j < 249, j + 5 <= 249)
    for j in range(246, 250):
        wait_scat(j % 8, j % 4)

    plsc.subcore_barrier()

    for k in range(NHOP):
        q = k % 4
        r0 = s * RPT + k * CH
        if k >= 4:
            pltpu.make_async_copy(
                rbuf[q], acc_hbm.at[c, pl.ds(s * RPT, CH)], wsem[q]).wait()
        pltpu.sync_copy(acc_sh.at[pl.ds(r0, CH)], rbuf[q])
        pltpu.async_copy(rbuf[q], acc_hbm.at[c, pl.ds(r0, CH)], wsem[q])
    for k in range(NHOP - 4, NHOP):
        q = k % 4
        pltpu.make_async_copy(
            rbuf[q], acc_hbm.at[c, pl.ds(s * RPT, CH)], wsem[q]).wait()


@functools.cache
def _scat_kernel():
    return pl.kernel(
        _scat_body,
        out_type=jax.ShapeDtypeStruct((2, NPAD, D), _f32),
        mesh=_mesh(),
        scratch_types=(
            [pltpu.VMEM((CH,), jnp.int32)] * 16
            + [pltpu.VMEM((CH, D), _f32)] * 4
            + [pltpu.VMEM_SHARED((NPAD, D), _f32)]
            + [pltpu.SemaphoreType.DMA] * 20
        ),
    )


def _scat_call(src, dst, hs):
    return _scat_kernel()(src, dst, hs)


_BLK = 1000
_DN = (((1,), (1,)), ((), ()))


def _prelu(v, a):
    return jnp.where(v >= 0, v, a * v)


def _k1_body(x_ref, lw_ref, lb_ref, a_ref, w1_ref, out_ref):
    a = a_ref[0, 0]
    h0 = lax.dot_general(x_ref[...], lw_ref[...], _DN,
                         preferred_element_type=_f32) + lb_ref[...]
    h0 = _prelu(h0, a)
    h1 = lax.dot_general(h0, w1_ref[...], _DN, preferred_element_type=_f32)
    out_ref[0] = h1[:, :D]
    out_ref[1] = h1[:, D:]


def _scale_body(h_ref, deg_ref, out_ref):
    dinv = lax.rsqrt(deg_ref[0] + deg_ref[1] + 1.0)
    out_ref[0] = h_ref[0] * dinv
    out_ref[1] = h_ref[1] * dinv


def _k3_body(acc_ref, hs_ref, deg_ref, b_ref, a_ref, w_ref, out_ref):
    a = a_ref[0, 0]
    dinv = lax.rsqrt(deg_ref[0] + deg_ref[1] + 1.0)
    m = jnp.concatenate([acc_ref[0] + hs_ref[0], acc_ref[1] + hs_ref[1]],
                        axis=1)
    o = _prelu(m * dinv + b_ref[...], a)
    h = lax.dot_general(o, w_ref[...], _DN, preferred_element_type=_f32)
    hs2 = h * dinv
    out_ref[0] = hs2[:, :D]
    out_ref[1] = hs2[:, D:]


def _k5_body(acc_ref, hs_ref, deg_ref, b_ref, a_ref, out_ref):
    a = a_ref[0, 0]
    dinv = lax.rsqrt(deg_ref[0] + deg_ref[1] + 1.0)
    m = jnp.concatenate([acc_ref[0] + hs_ref[0], acc_ref[1] + hs_ref[1]],
                        axis=1)
    out_ref[...] = _prelu(m * dinv + b_ref[...], a)


def _k1_call(x, lw, lb, a, w1):
    return pl.pallas_call(
        _k1_body,
        grid=(N // _BLK,),
        in_specs=[
            pl.BlockSpec((_BLK, 128), lambda i: (i, 0)),
            pl.BlockSpec((256, 128), lambda i: (0, 0)),
            pl.BlockSpec((1, 256), lambda i: (0, 0)),
            pl.BlockSpec((1, 1), lambda i: (0, 0)),
            pl.BlockSpec((256, 256), lambda i: (0, 0)),
        ],
        out_specs=pl.BlockSpec((2, _BLK, D), lambda i: (0, i, 0)),
        out_shape=jax.ShapeDtypeStruct((2, N, D), _f32),
    )(x, lw, lb, a, w1)


def _scale_call(h, deg3):
    return pl.pallas_call(
        _scale_body,
        grid=(N // _BLK,),
        in_specs=[
            pl.BlockSpec((2, _BLK, D), lambda i: (0, i, 0)),
            pl.BlockSpec((2, _BLK, 1), lambda i: (0, i, 0)),
        ],
        out_specs=pl.BlockSpec((2, _BLK, D), lambda i: (0, i, 0)),
        out_shape=jax.ShapeDtypeStruct((2, N, D), _f32),
    )(h, deg3)


def _k3_call(acc, hs, deg3, b, a, w):
    return pl.pallas_call(
        _k3_body,
        grid=(N // _BLK,),
        in_specs=[
            pl.BlockSpec((2, _BLK, D), lambda i: (0, i, 0)),
            pl.BlockSpec((2, _BLK, D), lambda i: (0, i, 0)),
            pl.BlockSpec((2, _BLK, 1), lambda i: (0, i, 0)),
            pl.BlockSpec((1, 256), lambda i: (0, 0)),
            pl.BlockSpec((1, 1), lambda i: (0, 0)),
            pl.BlockSpec((256, 256), lambda i: (0, 0)),
        ],
        out_specs=pl.BlockSpec((2, _BLK, D), lambda i: (0, i, 0)),
        out_shape=jax.ShapeDtypeStruct((2, N, D), _f32),
    )(acc, hs, deg3, b, a, w)


def _k5_call(acc, hs, deg3, b, a):
    return pl.pallas_call(
        _k5_body,
        grid=(N // _BLK,),
        in_specs=[
            pl.BlockSpec((2, _BLK, D), lambda i: (0, i, 0)),
            pl.BlockSpec((2, _BLK, D), lambda i: (0, i, 0)),
            pl.BlockSpec((2, _BLK, 1), lambda i: (0, i, 0)),
            pl.BlockSpec((1, 256), lambda i: (0, 0)),
            pl.BlockSpec((1, 1), lambda i: (0, 0)),
        ],
        out_specs=pl.BlockSpec((_BLK, 256), lambda i: (i, 0)),
        out_shape=jax.ShapeDtypeStruct((N, 256), _f32),
    )(acc, hs, deg3, b, a)


def kernel(x, edge_index, lin_W, lin_b, prelu_a, conv1_W, conv1_b,
           conv2_W, conv2_b):
    edge_index = edge_index.astype(jnp.int32)
    lb = lin_b.reshape(1, 256)
    b1 = conv1_b.reshape(1, 256)
    b2 = conv2_b.reshape(1, 256)
    a = prelu_a.reshape(1, 1)

    src = edge_index[0]
    dst = edge_index[1]
    h1 = _k1_call(x, lin_W, lb, a, conv1_W)
    deg3 = _deg_call(dst).reshape(2, N, 1)
    hs1 = _scale_call(h1, deg3)
    acc1 = _scat_call(src, dst, hs1.reshape(2 * N, D))
    hs2 = _k3_call(acc1, hs1, deg3, b1, a, conv2_W)
    acc2 = _scat_call(src, dst, hs2.reshape(2 * N, D))
    return _k5_call(acc2, hs2, deg3, b2, a)


# speedup vs baseline: 17.5084x; 1.0087x over previous
import functools

import jax
import jax.numpy as jnp
from jax import lax
from jax.experimental import pallas as pl
from jax.experimental.pallas import tpu as pltpu
from jax.experimental.pallas import tpu_sc as plsc

N = 10000
NPAD = 10240
E = 320000
D = 128
NS = 16
CH = 80
RPT = NPAD // NS
HOP = 128
_f32 = jnp.float32


def _mesh():
    return plsc.VectorSubcoreMesh(core_axis_name="c", subcore_axis_name="s")


_DEG_PER_TILE = (E // 2) // NS
_DEG_NCH = _DEG_PER_TILE // CH


def _deg_body(dst_hbm, deg_hbm, ib0, ib1, ib2, ib3, ones_v, zbuf, deg_sh,
              is0, is1, is2, is3, as0, as1):
    c = lax.axis_index("c")
    s = lax.axis_index("s")
    ebase = c * (E // 2) + s * _DEG_PER_TILE
    ibuf = (ib0, ib1, ib2, ib3)
    isem = (is0, is1, is2, is3)
    asem = (as0, as1)

    for k in range(CH // 16):
        ones_v[pl.ds(k * 16, 16)] = jnp.full((16,), 1.0, _f32)

    def zfill(i, carry):
        zbuf[pl.ds(i * 16, 16)] = jnp.zeros((16,), _f32)
        return carry

    lax.fori_loop(0, 2000 // 16, zfill, 0)

    def start_idx(j, q4):
        pltpu.async_copy(
            dst_hbm.at[pl.ds(ebase + j * CH, CH)], ibuf[q4], isem[q4])

    def wait_idx(q4):
        pltpu.make_async_copy(
            dst_hbm.at[pl.ds(ebase, CH)], ibuf[q4], isem[q4]).wait()

    def start_scat(q4, q2):
        pltpu.async_copy(ones_v, deg_sh.at[ibuf[q4]], asem[q2], add=True)

    def wait_scat(q4, q2):
        pltpu.make_async_copy(ones_v, deg_sh.at[ibuf[q4]], asem[q2]).wait()

    for j in range(2):
        start_idx(j, j)

    @pl.when(s < 5)
    def _():
        pltpu.sync_copy(zbuf, deg_sh.at[pl.ds(s * 2000, 2000)])

    plsc.subcore_barrier()

    def chunk(j, q4, first2, idx_ahead):
        q2 = q4 % 2
        wait_idx(q4)
        if not first2:
            wait_scat((q4 + 2) % 4, q2)
        if idx_ahead:
            start_idx(j + 2, (q4 + 2) % 4)
        start_scat(q4, q2)

    chunk(0, 0, True, True)
    chunk(1, 1, True, True)

    def group(g, carry):
        j0 = 2 + 4 * g
        for k in range(4):
            chunk(j0 + k, (2 + k) % 4, False, True)
        return carry

    lax.fori_loop(0, 30, group, 0)
    chunk(122, 122 % 4, False, True)
    chunk(123, 123 % 4, False, False)
    chunk(124, 124 % 4, False, False)
    for j in range(123, 125):
        wait_scat(j % 4, j % 2)

    plsc.subcore_barrier()

    @pl.when(s < 5)
    def _():
        pltpu.sync_copy(deg_sh.at[pl.ds(s * 2000, 2000)], zbuf)
        pltpu.sync_copy(zbuf, deg_hbm.at[pl.ds(c * N + s * 2000, 2000)])


@functools.cache
def _deg_kernel():
    return pl.kernel(
        _deg_body,
        out_type=jax.ShapeDtypeStruct((2 * N,), _f32),
        mesh=_mesh(),
        scratch_types=(
            [pltpu.VMEM((CH,), jnp.int32)] * 4
            + [pltpu.VMEM((CH,), _f32)]
            + [pltpu.VMEM((2000,), _f32)]
            + [pltpu.VMEM_SHARED((N,), _f32)]
            + [pltpu.SemaphoreType.DMA] * 6
        ),
    )


def _deg_call(dst):
    return _deg_kernel()(dst)


_SC_PER_TILE = E // NS
_SC_NCH = _SC_PER_TILE // CH


def _scat_body(src_hbm, dst_hbm, hs_hbm, acc_hbm,
               sb0, sb1, sb2, sb3, sb4, sb5, sb6, sb7,
               db0, db1, db2, db3, db4, db5, db6, db7,
               rb0, rb1, rb2, rb3, acc_sh,
               is0, is1, is2, is3, is4, is5, is6, is7,
               gs0, gs1, gs2, gs3, as0, as1, as2, as3,
               ws0, ws1, ws2, ws3):
    c = lax.axis_index("c")
    s = lax.axis_index("s")
    cN = c * N
    ebase = s * _SC_PER_TILE
    sbuf = (sb0, sb1, sb2, sb3, sb4, sb5, sb6, sb7)
    dbuf = (db0, db1, db2, db3, db4, db5, db6, db7)
    rbuf = (rb0, rb1, rb2, rb3)
    isem = (is0, is1, is2, is3, is4, is5, is6, is7)
    gsem = (gs0, gs1, gs2, gs3)
    asem = (as0, as1, as2, as3)
    wsem = (ws0, ws1, ws2, ws3)
    NHOP = RPT // CH

    def start_idx(j, q8):
        off = ebase + j * CH
        pltpu.async_copy(src_hbm.at[pl.ds(off, CH)], sbuf[q8], isem[q8])
        pltpu.async_copy(dst_hbm.at[pl.ds(off, CH)], dbuf[q8], isem[q8])

    def wait_idx(q8):
        pltpu.make_async_copy(
            src_hbm.at[pl.ds(ebase, CH)], sbuf[q8], isem[q8]).wait()
        pltpu.make_async_copy(
            dst_hbm.at[pl.ds(ebase, CH)], dbuf[q8], isem[q8]).wait()

    def fix_idx(q8):
        for k in range(CH // 16):
            v = sbuf[q8][pl.ds(k * 16, 16)]
            sbuf[q8][pl.ds(k * 16, 16)] = v + cN

    def start_gather(q8, q4):
        pltpu.async_copy(hs_hbm.at[sbuf[q8]], rbuf[q4], gsem[q4])

    def wait_gather(q8, q4):
        pltpu.make_async_copy(hs_hbm.at[sbuf[q8]], rbuf[q4], gsem[q4]).wait()

    def start_scat(q8, q4):
        pltpu.async_copy(rbuf[q4], acc_sh.at[dbuf[q8]], asem[q4], add=True)

    def wait_scat(q8, q4):
        pltpu.make_async_copy(
            rbuf[q4], acc_sh.at[dbuf[q8]], asem[q4]).wait()

    for j in range(5):
        start_idx(j, j)

    def zfill(i, carry):
        for j in range(8):
            rb3[i, pl.ds(j * 16, 16)] = jnp.zeros((16,), _f32)
        return carry

    lax.fori_loop(0, CH, zfill, 0)
    for k in range(NHOP):
        pltpu.async_copy(rb3, acc_sh.at[pl.ds(s * RPT + k * CH, CH)],
                         wsem[0])
    wait_idx(0)
    fix_idx(0)
    start_gather(0, 0)
    for k in range(NHOP):
        pltpu.make_async_copy(
            rb3, acc_sh.at[pl.ds(s * RPT, CH)], wsem[0]).wait()
    plsc.subcore_barrier()

    def chunk(j, q8, first3, has_next, idx_ahead):
        q4 = q8 % 4
        wait_gather(q8, q4)
        start_scat(q8, q4)
        if has_next:
            wait_idx((q8 + 1) % 8)
            fix_idx((q8 + 1) % 8)
            if not first3:
                wait_scat((q8 + 5) % 8, (q8 + 1) % 4)
            start_gather((q8 + 1) % 8, (q8 + 1) % 4)
        if idx_ahead:
            start_idx(j + 5, (q8 + 5) % 8)

    chunk(0, 0, True, True, True)
    chunk(1, 1, True, True, True)
    chunk(2, 2, True, True, True)

    def group(g, carry):
        j0 = 3 + 8 * g
        for k in range(8):
            chunk(j0 + k, (3 + k) % 8, False, True, True)
        return carry

    lax.fori_loop(0, 30, group, 0)
    for j in range(243, 250):
        chunk(j, j % 8, False, j < 249, j + 5 <= 249)
    for j in range(246, 250):
        wait_scat(j % 8, j % 4)

    plsc.subcore_barrier()

    for k in range(NHOP):
        q = k % 4
        r0 = s * RPT + k * CH
        if k >= 4:
            pltpu.make_async_copy(
                rbuf[q], acc_hbm.at[c, pl.ds(s * RPT, CH)], wsem[q]).wait()
        pltpu.sync_copy(acc_sh.at[pl.ds(r0, CH)], rbuf[q])
        pltpu.async_copy(rbuf[q], acc_hbm.at[c, pl.ds(r0, CH)], wsem[q])
    for k in range(NHOP - 4, NHOP):
        q = k % 4
        pltpu.make_async_copy(
            rbuf[q], acc_hbm.at[c, pl.ds(s * RPT, CH)], wsem[q]).wait()


@functools.cache
def _scat_kernel():
    return pl.kernel(
        _scat_body,
        out_type=jax.ShapeDtypeStruct((2, NPAD, D), _f32),
        mesh=_mesh(),
        scratch_types=(
            [pltpu.VMEM((CH,), jnp.int32)] * 16
            + [pltpu.VMEM((CH, D), _f32)] * 4
            + [pltpu.VMEM_SHARED((NPAD, D), _f32)]
            + [pltpu.SemaphoreType.DMA] * 20
        ),
    )


def _scat_call(src, dst, hs):
    return _scat_kernel()(src, dst, hs)


_BLK = 2000
_DN = (((1,), (1,)), ((), ()))


def _prelu(v, a):
    return jnp.where(v >= 0, v, a * v)


def _k1_body(x_ref, lw_ref, lb_ref, a_ref, w1_ref, out_ref):
    a = a_ref[0, 0]
    h0 = lax.dot_general(x_ref[...], lw_ref[...], _DN,
                         preferred_element_type=_f32) + lb_ref[...]
    h0 = _prelu(h0, a)
    h1 = lax.dot_general(h0, w1_ref[...], _DN, preferred_element_type=_f32)
    out_ref[0] = h1[:, :D]
    out_ref[1] = h1[:, D:]


def _scale_body(h_ref, deg_ref, out_ref):
    dinv = lax.rsqrt(deg_ref[0] + deg_ref[1] + 1.0)
    out_ref[0] = h_ref[0] * dinv
    out_ref[1] = h_ref[1] * dinv


def _k3_body(acc_ref, hs_ref, deg_ref, b_ref, a_ref, w_ref, out_ref):
    a = a_ref[0, 0]
    dinv = lax.rsqrt(deg_ref[0] + deg_ref[1] + 1.0)
    m = jnp.concatenate([acc_ref[0] + hs_ref[0], acc_ref[1] + hs_ref[1]],
                        axis=1)
    o = _prelu(m * dinv + b_ref[...], a)
    h = lax.dot_general(o, w_ref[...], _DN, preferred_element_type=_f32)
    hs2 = h * dinv
    out_ref[0] = hs2[:, :D]
    out_ref[1] = hs2[:, D:]


def _k5_body(acc_ref, hs_ref, deg_ref, b_ref, a_ref, out_ref):
    a = a_ref[0, 0]
    dinv = lax.rsqrt(deg_ref[0] + deg_ref[1] + 1.0)
    m = jnp.concatenate([acc_ref[0] + hs_ref[0], acc_ref[1] + hs_ref[1]],
                        axis=1)
    out_ref[...] = _prelu(m * dinv + b_ref[...], a)


def _k1_call(x, lw, lb, a, w1):
    return pl.pallas_call(
        _k1_body,
        grid=(N // _BLK,),
        in_specs=[
            pl.BlockSpec((_BLK, 128), lambda i: (i, 0)),
            pl.BlockSpec((256, 128), lambda i: (0, 0)),
            pl.BlockSpec((1, 256), lambda i: (0, 0)),
            pl.BlockSpec((1, 1), lambda i: (0, 0)),
            pl.BlockSpec((256, 256), lambda i: (0, 0)),
        ],
        out_specs=pl.BlockSpec((2, _BLK, D), lambda i: (0, i, 0)),
        out_shape=jax.ShapeDtypeStruct((2, N, D), _f32),
    )(x, lw, lb, a, w1)


def _scale_call(h, deg3):
    return pl.pallas_call(
        _scale_body,
        grid=(N // _BLK,),
        in_specs=[
            pl.BlockSpec((2, _BLK, D), lambda i: (0, i, 0)),
            pl.BlockSpec((2, _BLK, 1), lambda i: (0, i, 0)),
        ],
        out_specs=pl.BlockSpec((2, _BLK, D), lambda i: (0, i, 0)),
        out_shape=jax.ShapeDtypeStruct((2, N, D), _f32),
    )(h, deg3)


def _k3_call(acc, hs, deg3, b, a, w):
    return pl.pallas_call(
        _k3_body,
        grid=(N // _BLK,),
        in_specs=[
            pl.BlockSpec((2, _BLK, D), lambda i: (0, i, 0)),
            pl.BlockSpec((2, _BLK, D), lambda i: (0, i, 0)),
            pl.BlockSpec((2, _BLK, 1), lambda i: (0, i, 0)),
            pl.BlockSpec((1, 256), lambda i: (0, 0)),
            pl.BlockSpec((1, 1), lambda i: (0, 0)),
            pl.BlockSpec((256, 256), lambda i: (0, 0)),
        ],
        out_specs=pl.BlockSpec((2, _BLK, D), lambda i: (0, i, 0)),
        out_shape=jax.ShapeDtypeStruct((2, N, D), _f32),
    )(acc, hs, deg3, b, a, w)


def _k5_call(acc, hs, deg3, b, a):
    return pl.pallas_call(
        _k5_body,
        grid=(N // _BLK,),
        in_specs=[
            pl.BlockSpec((2, _BLK, D), lambda i: (0, i, 0)),
            pl.BlockSpec((2, _BLK, D), lambda i: (0, i, 0)),
            pl.BlockSpec((2, _BLK, 1), lambda i: (0, i, 0)),
            pl.BlockSpec((1, 256), lambda i: (0, 0)),
            pl.BlockSpec((1, 1), lambda i: (0, 0)),
        ],
        out_specs=pl.BlockSpec((_BLK, 256), lambda i: (i, 0)),
        out_shape=jax.ShapeDtypeStruct((N, 256), _f32),
    )(acc, hs, deg3, b, a)


def kernel(x, edge_index, lin_W, lin_b, prelu_a, conv1_W, conv1_b,
           conv2_W, conv2_b):
    edge_index = edge_index.astype(jnp.int32)
    lb = lin_b.reshape(1, 256)
    b1 = conv1_b.reshape(1, 256)
    b2 = conv2_b.reshape(1, 256)
    a = prelu_a.reshape(1, 1)

    src = edge_index[0]
    dst = edge_index[1]
    h1 = _k1_call(x, lin_W, lb, a, conv1_W)
    deg3 = _deg_call(dst).reshape(2, N, 1)
    hs1 = _scale_call(h1, deg3)
    acc1 = _scat_call(src, dst, hs1.reshape(2 * N, D))
    hs2 = _k3_call(acc1, hs1, deg3, b1, a, conv2_W)
    acc2 = _scat_call(src, dst, hs2.reshape(2 * N, D))
    return _k5_call(acc2, hs2, deg3, b2, a)


# speedup vs baseline: 17.5139x; 1.0003x over previous
import functools

import jax
import jax.numpy as jnp
from jax import lax
from jax.experimental import pallas as pl
from jax.experimental.pallas import tpu as pltpu
from jax.experimental.pallas import tpu_sc as plsc

N = 10000
NPAD = 10240
E = 320000
D = 128
NS = 16
CH = 80
RPT = NPAD // NS
HOP = 128
_f32 = jnp.float32


def _mesh():
    return plsc.VectorSubcoreMesh(core_axis_name="c", subcore_axis_name="s")


_DEG_PER_TILE = (E // 2) // NS
_DEG_NCH = _DEG_PER_TILE // CH


def _deg_body(dst_hbm, deg_hbm, ib0, ib1, ib2, ib3, ones_v, zbuf, deg_sh,
              is0, is1, is2, is3, as0, as1):
    c = lax.axis_index("c")
    s = lax.axis_index("s")
    ebase = c * (E // 2) + s * _DEG_PER_TILE
    ibuf = (ib0, ib1, ib2, ib3)
    isem = (is0, is1, is2, is3)
    asem = (as0, as1)

    for k in range(CH // 16):
        ones_v[pl.ds(k * 16, 16)] = jnp.full((16,), 1.0, _f32)

    def zfill(i, carry):
        zbuf[pl.ds(i * 16, 16)] = jnp.zeros((16,), _f32)
        return carry

    lax.fori_loop(0, 2000 // 16, zfill, 0)

    def start_idx(j, q4):
        pltpu.async_copy(
            dst_hbm.at[pl.ds(ebase + j * CH, CH)], ibuf[q4], isem[q4])

    def wait_idx(q4):
        pltpu.make_async_copy(
            dst_hbm.at[pl.ds(ebase, CH)], ibuf[q4], isem[q4]).wait()

    def start_scat(q4, q2):
        pltpu.async_copy(ones_v, deg_sh.at[ibuf[q4]], asem[q2], add=True)

    def wait_scat(q4, q2):
        pltpu.make_async_copy(ones_v, deg_sh.at[ibuf[q4]], asem[q2]).wait()

    for j in range(2):
        start_idx(j, j)

    @pl.when(s < 5)
    def _():
        pltpu.sync_copy(zbuf, deg_sh.at[pl.ds(s * 2000, 2000)])

    plsc.subcore_barrier()

    def chunk(j, q4, first2, idx_ahead):
        q2 = q4 % 2
        wait_idx(q4)
        if not first2:
            wait_scat((q4 + 2) % 4, q2)
        if idx_ahead:
            start_idx(j + 2, (q4 + 2) % 4)
        start_scat(q4, q2)

    chunk(0, 0, True, True)
    chunk(1, 1, True, True)

    def group(g, carry):
        j0 = 2 + 4 * g
        for k in range(4):
            chunk(j0 + k, (2 + k) % 4, False, True)
        return carry

    lax.fori_loop(0, 30, group, 0)
    chunk(122, 122 % 4, False, True)
    chunk(123, 123 % 4, False, False)
    chunk(124, 124 % 4, False, False)
    for j in range(123, 125):
        wait_scat(j % 4, j % 2)

    plsc.subcore_barrier()

    @pl.when(s < 5)
    def _():
        pltpu.sync_copy(deg_sh.at[pl.ds(s * 2000, 2000)], zbuf)
        pltpu.sync_copy(zbuf, deg_hbm.at[pl.ds(c * N + s * 2000, 2000)])


@functools.cache
def _deg_kernel():
    return pl.kernel(
        _deg_body,
        out_type=jax.ShapeDtypeStruct((2 * N,), _f32),
        mesh=_mesh(),
        scratch_types=(
            [pltpu.VMEM((CH,), jnp.int32)] * 4
            + [pltpu.VMEM((CH,), _f32)]
            + [pltpu.VMEM((2000,), _f32)]
            + [pltpu.VMEM_SHARED((N,), _f32)]
            + [pltpu.SemaphoreType.DMA] * 6
        ),
    )


def _deg_call(dst):
    return _deg_kernel()(dst)


_SC_PER_TILE = E // NS
_SC_NCH = _SC_PER_TILE // CH


def _scat_body(src_hbm, dst_hbm, hs_hbm, acc_hbm,
               sb0, sb1, sb2, sb3, sb4, sb5, sb6, sb7,
               db0, db1, db2, db3, db4, db5, db6, db7,
               rb0, rb1, rb2, rb3, acc_sh,
               is0, is1, is2, is3, is4, is5, is6, is7,
               gs0, gs1, gs2, gs3, as0, as1, as2, as3,
               ws0, ws1, ws2, ws3):
    c = lax.axis_index("c")
    s = lax.axis_index("s")
    cN = c * N
    ebase = s * _SC_PER_TILE
    sbuf = (sb0, sb1, sb2, sb3, sb4, sb5, sb6, sb7)
    dbuf = (db0, db1, db2, db3, db4, db5, db6, db7)
    rbuf = (rb0, rb1, rb2, rb3)
    isem = (is0, is1, is2, is3, is4, is5, is6, is7)
    gsem = (gs0, gs1, gs2, gs3)
    asem = (as0, as1, as2, as3)
    wsem = (ws0, ws1, ws2, ws3)
    NHOP = RPT // CH

    def start_idx(j, q8):
        off = ebase + j * CH
        pltpu.async_copy(src_hbm.at[pl.ds(off, CH)], sbuf[q8], isem[q8])
        pltpu.async_copy(dst_hbm.at[pl.ds(off, CH)], dbuf[q8], isem[q8])

    def wait_idx(q8):
        pltpu.make_async_copy(
            src_hbm.at[pl.ds(ebase, CH)], sbuf[q8], isem[q8]).wait()
        pltpu.make_async_copy(
            dst_hbm.at[pl.ds(ebase, CH)], dbuf[q8], isem[q8]).wait()

    def fix_idx(q8):
        for k in range(CH // 16):
            v = sbuf[q8][pl.ds(k * 16, 16)]
            sbuf[q8][pl.ds(k * 16, 16)] = v + cN

    def start_gather(q8, q4):
        pltpu.async_copy(hs_hbm.at[sbuf[q8]], rbuf[q4], gsem[q4])

    def wait_gather(q8, q4):
        pltpu.make_async_copy(hs_hbm.at[sbuf[q8]], rbuf[q4], gsem[q4]).wait()

    def start_scat(q8, q4):
        pltpu.async_copy(rbuf[q4], acc_sh.at[dbuf[q8]], asem[q4], add=True)

    def wait_scat(q8, q4):
        pltpu.make_async_copy(
            rbuf[q4], acc_sh.at[dbuf[q8]], asem[q4]).wait()

    for j in range(5):
        start_idx(j, j)

    def zfill(i, carry):
        for j in range(8):
            rb3[i, pl.ds(j * 16, 16)] = jnp.zeros((16,), _f32)
        return carry

    lax.fori_loop(0, CH, zfill, 0)
    for k in range(NHOP):
        pltpu.async_copy(rb3, acc_sh.at[pl.ds(s * RPT + k * CH, CH)],
                         wsem[0])
    wait_idx(0)
    fix_idx(0)
    start_gather(0, 0)
    for k in range(NHOP):
        pltpu.make_async_copy(
            rb3, acc_sh.at[pl.ds(s * RPT, CH)], wsem[0]).wait()
    plsc.subcore_barrier()

    def chunk(j, q8, first3, has_next, idx_ahead):
        q4 = q8 % 4
        wait_gather(q8, q4)
        start_scat(q8, q4)
        if has_next:
            wait_idx((q8 + 1) % 8)
            fix_idx((q8 + 1) % 8)
            if not first3:
                wait_scat((q8 + 5) % 8, (q8 + 1) % 4)
            start_gather((q8 + 1) % 8, (q8 + 1) % 4)
        if idx_ahead:
            start_idx(j + 5, (q8 + 5) % 8)

    chunk(0, 0, True, True, True)
    chunk(1, 1, True, True, True)
    chunk(2, 2, True, True, True)

    def group(g, carry):
        j0 = 3 + 8 * g
        for k in range(8):
            chunk(j0 + k, (3 + k) % 8, False, True, True)
        return carry

    lax.fori_loop(0, 30, group, 0)
    for j in range(243, 250):
        chunk(j, j % 8, False, j < 249, j + 5 <= 249)
    for j in range(246, 250):
        wait_scat(j % 8, j % 4)

    plsc.subcore_barrier()

    for k in range(NHOP):
        q = k % 4
        r0 = s * RPT + k * CH
        if k >= 4:
            pltpu.make_async_copy(
                rbuf[q], acc_hbm.at[c, pl.ds(s * RPT, CH)], wsem[q]).wait()
        pltpu.sync_copy(acc_sh.at[pl.ds(r0, CH)], rbuf[q])
        pltpu.async_copy(rbuf[q], acc_hbm.at[c, pl.ds(r0, CH)], wsem[q])
    for k in range(NHOP - 4, NHOP):
        q = k % 4
        pltpu.make_async_copy(
            rbuf[q], acc_hbm.at[c, pl.ds(s * RPT, CH)], wsem[q]).wait()


@functools.cache
def _scat_kernel():
    return pl.kernel(
        _scat_body,
        out_type=jax.ShapeDtypeStruct((2, NPAD, D), _f32),
        mesh=_mesh(),
        scratch_types=(
            [pltpu.VMEM((CH,), jnp.int32)] * 16
            + [pltpu.VMEM((CH, D), _f32)] * 4
            + [pltpu.VMEM_SHARED((NPAD, D), _f32)]
            + [pltpu.SemaphoreType.DMA] * 20
        ),
    )


def _scat_call(src, dst, hs):
    return _scat_kernel()(src, dst, hs)


_BLK = 2000
_DN = (((1,), (1,)), ((), ()))


def _prelu(v, a):
    return jnp.where(v >= 0, v, a * v)


def _k1_body(x_ref, lw_ref, lb_ref, a_ref, w1_ref, deg_ref, out_ref):
    a = a_ref[0, 0]
    h0 = lax.dot_general(x_ref[...], lw_ref[...], _DN,
                         preferred_element_type=_f32) + lb_ref[...]
    h0 = _prelu(h0, a)
    h1 = lax.dot_general(h0, w1_ref[...], _DN, preferred_element_type=_f32)
    dinv = lax.rsqrt(deg_ref[0] + deg_ref[1] + 1.0)
    hs = h1 * dinv
    out_ref[0] = hs[:, :D]
    out_ref[1] = hs[:, D:]


def _k3_body(acc_ref, hs_ref, deg_ref, b_ref, a_ref, w_ref, out_ref):
    a = a_ref[0, 0]
    dinv = lax.rsqrt(deg_ref[0] + deg_ref[1] + 1.0)
    m = jnp.concatenate([acc_ref[0] + hs_ref[0], acc_ref[1] + hs_ref[1]],
                        axis=1)
    o = _prelu(m * dinv + b_ref[...], a)
    h = lax.dot_general(o, w_ref[...], _DN, preferred_element_type=_f32)
    hs2 = h * dinv
    out_ref[0] = hs2[:, :D]
    out_ref[1] = hs2[:, D:]


def _k5_body(acc_ref, hs_ref, deg_ref, b_ref, a_ref, out_ref):
    a = a_ref[0, 0]
    dinv = lax.rsqrt(deg_ref[0] + deg_ref[1] + 1.0)
    m = jnp.concatenate([acc_ref[0] + hs_ref[0], acc_ref[1] + hs_ref[1]],
                        axis=1)
    out_ref[...] = _prelu(m * dinv + b_ref[...], a)


def _k1_call(x, lw, lb, a, w1, deg3):
    return pl.pallas_call(
        _k1_body,
        grid=(N // _BLK,),
        in_specs=[
            pl.BlockSpec((_BLK, 128), lambda i: (i, 0)),
            pl.BlockSpec((256, 128), lambda i: (0, 0)),
            pl.BlockSpec((1, 256), lambda i: (0, 0)),
            pl.BlockSpec((1, 1), lambda i: (0, 0)),
            pl.BlockSpec((256, 256), lambda i: (0, 0)),
            pl.BlockSpec((2, _BLK, 1), lambda i: (0, i, 0)),
        ],
        out_specs=pl.BlockSpec((2, _BLK, D), lambda i: (0, i, 0)),
        out_shape=jax.ShapeDtypeStruct((2, N, D), _f32),
    )(x, lw, lb, a, w1, deg3)


def _k3_call(acc, hs, deg3, b, a, w):
    return pl.pallas_call(
        _k3_body,
        grid=(N // _BLK,),
        in_specs=[
            pl.BlockSpec((2, _BLK, D), lambda i: (0, i, 0)),
            pl.BlockSpec((2, _BLK, D), lambda i: (0, i, 0)),
            pl.BlockSpec((2, _BLK, 1), lambda i: (0, i, 0)),
            pl.BlockSpec((1, 256), lambda i: (0, 0)),
            pl.BlockSpec((1, 1), lambda i: (0, 0)),
            pl.BlockSpec((256, 256), lambda i: (0, 0)),
        ],
        out_specs=pl.BlockSpec((2, _BLK, D), lambda i: (0, i, 0)),
        out_shape=jax.ShapeDtypeStruct((2, N, D), _f32),
    )(acc, hs, deg3, b, a, w)


def _k5_call(acc, hs, deg3, b, a):
    return pl.pallas_call(
        _k5_body,
        grid=(N // _BLK,),
        in_specs=[
            pl.BlockSpec((2, _BLK, D), lambda i: (0, i, 0)),
            pl.BlockSpec((2, _BLK, D), lambda i: (0, i, 0)),
            pl.BlockSpec((2, _BLK, 1), lambda i: (0, i, 0)),
            pl.BlockSpec((1, 256), lambda i: (0, 0)),
            pl.BlockSpec((1, 1), lambda i: (0, 0)),
        ],
        out_specs=pl.BlockSpec((_BLK, 256), lambda i: (i, 0)),
        out_shape=jax.ShapeDtypeStruct((N, 256), _f32),
    )(acc, hs, deg3, b, a)


def kernel(x, edge_index, lin_W, lin_b, prelu_a, conv1_W, conv1_b,
           conv2_W, conv2_b):
    edge_index = edge_index.astype(jnp.int32)
    lb = lin_b.reshape(1, 256)
    b1 = conv1_b.reshape(1, 256)
    b2 = conv2_b.reshape(1, 256)
    a = prelu_a.reshape(1, 1)

    src = edge_index[0]
    dst = edge_index[1]
    deg3 = _deg_call(dst).reshape(2, N, 1)
    hs1 = _k1_call(x, lin_W, lb, a, conv1_W, deg3)
    acc1 = _scat_call(src, dst, hs1.reshape(2 * N, D))
    hs2 = _k3_call(acc1, hs1, deg3, b1, a, conv2_W)
    acc2 = _scat_call(src, dst, hs2.reshape(2 * N, D))
    return _k5_call(acc2, hs2, deg3, b2, a)


# speedup vs baseline: 17.5407x; 1.0015x over previous
import functools

import jax
import jax.numpy as jnp
from jax import lax
from jax.experimental import pallas as pl
from jax.experimental.pallas import tpu as pltpu
from jax.experimental.pallas import tpu_sc as plsc

N = 10000
NPAD = 10240
E = 320000
D = 128
NS = 16
CH = 80
RPT = NPAD // NS
_f32 = jnp.float32


def _mesh():
    return plsc.VectorSubcoreMesh(core_axis_name="c", subcore_axis_name="s")


_DEG_PER_TILE = (E // 2) // NS
_DEG_NCH = _DEG_PER_TILE // CH


def _deg_body(dst_hbm, deg_hbm, ib0, ib1, ib2, ib3, ones_v, zbuf, deg_sh,
              is0, is1, is2, is3, as0, as1):
    c = lax.axis_index("c")
    s = lax.axis_index("s")
    ebase = c * (E // 2) + s * _DEG_PER_TILE
    ibuf = (ib0, ib1, ib2, ib3)
    isem = (is0, is1, is2, is3)
    asem = (as0, as1)

    for k in range(CH // 16):
        ones_v[pl.ds(k * 16, 16)] = jnp.full((16,), 1.0, _f32)

    def zfill(i, carry):
        zbuf[pl.ds(i * 16, 16)] = jnp.zeros((16,), _f32)
        return carry

    lax.fori_loop(0, 2000 // 16, zfill, 0)

    def start_idx(j, q4):
        pltpu.async_copy(
            dst_hbm.at[pl.ds(ebase + j * CH, CH)], ibuf[q4], isem[q4])

    def wait_idx(q4):
        pltpu.make_async_copy(
            dst_hbm.at[pl.ds(ebase, CH)], ibuf[q4], isem[q4]).wait()

    def start_scat(q4, q2):
        pltpu.async_copy(ones_v, deg_sh.at[ibuf[q4]], asem[q2], add=True)

    def wait_scat(q4, q2):
        pltpu.make_async_copy(ones_v, deg_sh.at[ibuf[q4]], asem[q2]).wait()

    for j in range(2):
        start_idx(j, j)

    @pl.when(s < 5)
    def _():
        pltpu.sync_copy(zbuf, deg_sh.at[pl.ds(s * 2000, 2000)])

    plsc.subcore_barrier()

    def chunk(j, q4, first2, idx_ahead):
        q2 = q4 % 2
        wait_idx(q4)
        if not first2:
            wait_scat((q4 + 2) % 4, q2)
        if idx_ahead:
            start_idx(j + 2, (q4 + 2) % 4)
        start_scat(q4, q2)

    chunk(0, 0, True, True)
    chunk(1, 1, True, True)

    def group(g, carry):
        j0 = 2 + 4 * g
        for k in range(4):
            chunk(j0 + k, (2 + k) % 4, False, True)
        return carry

    lax.fori_loop(0, 30, group, 0)
    chunk(122, 122 % 4, False, True)
    chunk(123, 123 % 4, False, False)
    chunk(124, 124 % 4, False, False)
    for j in range(123, 125):
        wait_scat(j % 4, j % 2)

    plsc.subcore_barrier()

    @pl.when(s < 5)
    def _():
        pltpu.sync_copy(deg_sh.at[pl.ds(s * 2000, 2000)], zbuf)
        pltpu.sync_copy(zbuf, deg_hbm.at[pl.ds(c * N + s * 2000, 2000)])


@functools.cache
def _deg_kernel():
    return pl.kernel(
        _deg_body,
        out_type=jax.ShapeDtypeStruct((2 * N,), _f32),
        mesh=_mesh(),
        scratch_types=(
            [pltpu.VMEM((CH,), jnp.int32)] * 4
            + [pltpu.VMEM((CH,), _f32)]
            + [pltpu.VMEM((2000,), _f32)]
            + [pltpu.VMEM_SHARED((N,), _f32)]
            + [pltpu.SemaphoreType.DMA] * 6
        ),
    )


def _deg_call(dst):
    return _deg_kernel()(dst)


_SC_PER_TILE = E // NS
_SC_NCH = _SC_PER_TILE // CH


def _scat_body(src_hbm, dst_hbm, hs_hbm, acc_hbm,
               sb0, sb1, sb2, sb3, sb4, sb5, sb6, sb7,
               db0, db1, db2, db3, db4, db5, db6, db7,
               rb0, rb1, rb2, rb3, acc_sh,
               is0, is1, is2, is3, is4, is5, is6, is7,
               gs0, gs1, gs2, gs3, as0, as1, as2, as3,
               ws0, ws1, ws2, ws3):
    c = lax.axis_index("c")
    s = lax.axis_index("s")
    cN = c * N
    ebase = s * _SC_PER_TILE
    sbuf = (sb0, sb1, sb2, sb3, sb4, sb5, sb6, sb7)
    dbuf = (db0, db1, db2, db3, db4, db5, db6, db7)
    rbuf = (rb0, rb1, rb2, rb3)
    isem = (is0, is1, is2, is3, is4, is5, is6, is7)
    gsem = (gs0, gs1, gs2, gs3)
    asem = (as0, as1, as2, as3)
    wsem = (ws0, ws1, ws2, ws3)
    NHOP = RPT // CH

    def start_idx(j, q8):
        off = ebase + j * CH
        pltpu.async_copy(src_hbm.at[pl.ds(off, CH)], sbuf[q8], isem[q8])
        pltpu.async_copy(dst_hbm.at[pl.ds(off, CH)], dbuf[q8], isem[q8])

    def wait_idx(q8):
        pltpu.make_async_copy(
            src_hbm.at[pl.ds(ebase, CH)], sbuf[q8], isem[q8]).wait()
        pltpu.make_async_copy(
            dst_hbm.at[pl.ds(ebase, CH)], dbuf[q8], isem[q8]).wait()

    def fix_idx(q8):
        for k in range(CH // 16):
            v = sbuf[q8][pl.ds(k * 16, 16)]
            sbuf[q8][pl.ds(k * 16, 16)] = v + cN

    def start_gather(q8, q4):
        pltpu.async_copy(hs_hbm.at[sbuf[q8]], rbuf[q4], gsem[q4])

    def wait_gather(q8, q4):
        pltpu.make_async_copy(hs_hbm.at[sbuf[q8]], rbuf[q4], gsem[q4]).wait()

    def start_scat(q8, q4):
        pltpu.async_copy(rbuf[q4], acc_sh.at[dbuf[q8]], asem[q4], add=True)

    def wait_scat(q8, q4):
        pltpu.make_async_copy(
            rbuf[q4], acc_sh.at[dbuf[q8]], asem[q4]).wait()

    for j in range(5):
        start_idx(j, j)

    def zfill(i, carry):
        for j in range(8):
            rb3[i, pl.ds(j * 16, 16)] = jnp.zeros((16,), _f32)
        return carry

    lax.fori_loop(0, CH, zfill, 0)
    for k in range(NHOP):
        pltpu.async_copy(rb3, acc_sh.at[pl.ds(s * RPT + k * CH, CH)],
                         wsem[0])
    wait_idx(0)
    fix_idx(0)
    start_gather(0, 0)
    for k in range(NHOP):
        pltpu.make_async_copy(
            rb3, acc_sh.at[pl.ds(s * RPT, CH)], wsem[0]).wait()
    plsc.subcore_barrier()

    def chunk(j, q8, first3, has_next, idx_ahead):
        q4 = q8 % 4
        wait_gather(q8, q4)
        start_scat(q8, q4)
        if has_next:
            wait_idx((q8 + 1) % 8)
            fix_idx((q8 + 1) % 8)
            if not first3:
                wait_scat((q8 + 5) % 8, (q8 + 1) % 4)
            start_gather((q8 + 1) % 8, (q8 + 1) % 4)
        if idx_ahead:
            start_idx(j + 5, (q8 + 5) % 8)

    chunk(0, 0, True, True, True)
    chunk(1, 1, True, True, True)
    chunk(2, 2, True, True, True)

    def group(g, carry):
        j0 = 3 + 8 * g
        for k in range(8):
            chunk(j0 + k, (3 + k) % 8, False, True, True)
        return carry

    lax.fori_loop(0, 30, group, 0)
    for j in range(243, 250):
        chunk(j, j % 8, False, j < 249, j + 5 <= 249)
    for j in range(246, 250):
        wait_scat(j % 8, j % 4)

    plsc.subcore_barrier()

    for k in range(NHOP):
        q = k % 4
        r0 = s * RPT + k * CH
        if k >= 4:
            pltpu.make_async_copy(
                rbuf[q], acc_hbm.at[c, pl.ds(s * RPT, CH)], wsem[q]).wait()
        pltpu.sync_copy(acc_sh.at[pl.ds(r0, CH)], rbuf[q])
        pltpu.async_copy(rbuf[q], acc_hbm.at[c, pl.ds(r0, CH)], wsem[q])
    for k in range(NHOP - 4, NHOP):
        q = k % 4
        pltpu.make_async_copy(
            rbuf[q], acc_hbm.at[c, pl.ds(s * RPT, CH)], wsem[q]).wait()


@functools.cache
def _scat_kernel():
    return pl.kernel(
        _scat_body,
        out_type=jax.ShapeDtypeStruct((2, NPAD, D), _f32),
        mesh=_mesh(),
        scratch_types=(
            [pltpu.VMEM((CH,), jnp.int32)] * 16
            + [pltpu.VMEM((CH, D), _f32)] * 4
            + [pltpu.VMEM_SHARED((NPAD, D), _f32)]
            + [pltpu.SemaphoreType.DMA] * 20
        ),
    )


def _scat_call(src, dst, hs):
    return _scat_kernel()(src, dst, hs)


_BLK = 2000
_DN = (((1,), (1,)), ((), ()))


def _prelu(v, a):
    return jnp.where(v >= 0, v, a * v)


def _k1_body(x_ref, lw_ref, lb_ref, a_ref, w1_ref, deg_ref, out_ref):
    a = a_ref[0, 0]
    h0 = lax.dot_general(x_ref[...], lw_ref[...], _DN,
                         preferred_element_type=_f32) + lb_ref[...]
    h0 = _prelu(h0, a)
    h1 = lax.dot_general(h0, w1_ref[...], _DN, preferred_element_type=_f32)
    dinv = lax.rsqrt(deg_ref[0] + deg_ref[1] + 1.0)
    hs = h1 * dinv
    out_ref[0] = hs[:, :D]
    out_ref[1] = hs[:, D:]


def _k3_body(acc_ref, hs_ref, deg_ref, b_ref, a_ref, w_ref, out_ref):
    a = a_ref[0, 0]
    dinv = lax.rsqrt(deg_ref[0] + deg_ref[1] + 1.0)
    m = jnp.concatenate([acc_ref[0] + hs_ref[0], acc_ref[1] + hs_ref[1]],
                        axis=1)
    o = _prelu(m * dinv + b_ref[...], a)
    h = lax.dot_general(o, w_ref[...], _DN, preferred_element_type=_f32)
    hs2 = h * dinv
    out_ref[0] = hs2[:, :D]
    out_ref[1] = hs2[:, D:]


def _k5_body(acc_ref, hs_ref, deg_ref, b_ref, a_ref, out_ref):
    a = a_ref[0, 0]
    dinv = lax.rsqrt(deg_ref[0] + deg_ref[1] + 1.0)
    m = jnp.concatenate([acc_ref[0] + hs_ref[0], acc_ref[1] + hs_ref[1]],
                        axis=1)
    out_ref[...] = _prelu(m * dinv + b_ref[...], a)


def _k1_call(x, lw, lb, a, w1, deg3):
    return pl.pallas_call(
        _k1_body,
        grid=(N // _BLK,),
        in_specs=[
            pl.BlockSpec((_BLK, 128), lambda i: (i, 0)),
            pl.BlockSpec((256, 128), lambda i: (0, 0)),
            pl.BlockSpec((1, 256), lambda i: (0, 0)),
            pl.BlockSpec((1, 1), lambda i: (0, 0)),
            pl.BlockSpec((256, 256), lambda i: (0, 0)),
            pl.BlockSpec((2, _BLK, 1), lambda i: (0, i, 0)),
        ],
        out_specs=pl.BlockSpec((2, _BLK, D), lambda i: (0, i, 0)),
        out_shape=jax.ShapeDtypeStruct((2, N, D), _f32),
    )(x, lw, lb, a, w1, deg3)


def _k3_call(acc, hs, deg3, b, a, w):
    return pl.pallas_call(
        _k3_body,
        grid=(N // _BLK,),
        in_specs=[
            pl.BlockSpec((2, _BLK, D), lambda i: (0, i, 0)),
            pl.BlockSpec((2, _BLK, D), lambda i: (0, i, 0)),
            pl.BlockSpec((2, _BLK, 1), lambda i: (0, i, 0)),
            pl.BlockSpec((1, 256), lambda i: (0, 0)),
            pl.BlockSpec((1, 1), lambda i: (0, 0)),
            pl.BlockSpec((256, 256), lambda i: (0, 0)),
        ],
        out_specs=pl.BlockSpec((2, _BLK, D), lambda i: (0, i, 0)),
        out_shape=jax.ShapeDtypeStruct((2, N, D), _f32),
    )(acc, hs, deg3, b, a, w)


def _k5_call(acc, hs, deg3, b, a):
    return pl.pallas_call(
        _k5_body,
        grid=(N // _BLK,),
        in_specs=[
            pl.BlockSpec((2, _BLK, D), lambda i: (0, i, 0)),
            pl.BlockSpec((2, _BLK, D), lambda i: (0, i, 0)),
            pl.BlockSpec((2, _BLK, 1), lambda i: (0, i, 0)),
            pl.BlockSpec((1, 256), lambda i: (0, 0)),
            pl.BlockSpec((1, 1), lambda i: (0, 0)),
        ],
        out_specs=pl.BlockSpec((_BLK, 256), lambda i: (i, 0)),
        out_shape=jax.ShapeDtypeStruct((N, 256), _f32),
    )(acc, hs, deg3, b, a)


def kernel(x, edge_index, lin_W, lin_b, prelu_a, conv1_W, conv1_b,
           conv2_W, conv2_b):
    edge_index = edge_index.astype(jnp.int32)
    lb = lin_b.reshape(1, 256)
    b1 = conv1_b.reshape(1, 256)
    b2 = conv2_b.reshape(1, 256)
    a = prelu_a.reshape(1, 1)

    src = edge_index[0]
    dst = edge_index[1]
    deg3 = _deg_call(dst).reshape(2, N, 1)
    hs1 = _k1_call(x, lin_W, lb, a, conv1_W, deg3)
    acc1 = _scat_call(src, dst, hs1.reshape(2 * N, D))
    hs2 = _k3_call(acc1, hs1, deg3, b1, a, conv2_W)
    acc2 = _scat_call(src, dst, hs2.reshape(2 * N, D))
    return _k5_call(acc2, hs2, deg3, b2, a)
